# baseline (speedup 1.0000x reference)
import functools

import jax
import jax.numpy as jnp
from jax import lax
from jax.experimental import pallas as pl
from jax.experimental.pallas import tpu as pltpu
from jax.experimental.pallas import tpu_sc as plsc

_GRID_SIZE = 5
_SPLINE_ORDER = 3
_NB = _GRID_SIZE + _SPLINE_ORDER
_H = 2.0 / _GRID_SIZE
_G0 = -1.0 - _SPLINE_ORDER * _H
_NKNOTS = _GRID_SIZE + 2 * _SPLINE_ORDER + 1
_EPS = 1e-5


def _bspline_bases(y):
    g = [_G0 + _H * j for j in range(_NKNOTS)]
    b = [((y >= g[j]) & (y < g[j + 1])).astype(y.dtype) for j in range(_NKNOTS - 1)]
    for k in range(1, _SPLINE_ORDER + 1):
        inv = 1.0 / (_H * k)
        nb = []
        for j in range(len(b) - 1):
            left = (y - g[j]) * inv
            right = (g[j + k + 1] - y) * inv
            nb.append(left * b[j] + right * b[j + 1])
        b = nb
    return b


def _kan_core(y, bwt_ref, sw_ref):
    acc = jnp.dot(y * jax.nn.sigmoid(y), bwt_ref[...],
                  preferred_element_type=jnp.float32)
    bases = _bspline_bases(y)
    for b in range(_NB):
        acc = acc + jnp.dot(bases[b], sw_ref[b],
                            preferred_element_type=jnp.float32)
    return acc


def _accum_stats(stats_ref, h):
    st = jnp.concatenate(
        [jnp.sum(h, axis=0, keepdims=True),
         jnp.sum(h * h, axis=0, keepdims=True)], axis=0)

    @pl.when(pl.program_id(0) == 0)
    def _():
        stats_ref[...] = st

    @pl.when(pl.program_id(0) != 0)
    def _():
        stats_ref[...] = stats_ref[...] + st


def _bn_affine(stats_ref, gamma_ref, beta_ref, n):
    mu = stats_ref[0:1, :] * (1.0 / n)
    ex2 = stats_ref[1:2, :] * (1.0 / n)
    var = ex2 - mu * mu
    scale = gamma_ref[...] * lax.rsqrt(var + _EPS)
    shift = beta_ref[...] - mu * scale
    return scale, shift


def _layer1_body(x_ref, aa_ref, ab_ref, bwt_ref, sw_ref, h_ref, stats_ref):
    y = x_ref[...] + aa_ref[0] + ab_ref[0]
    h = _kan_core(y, bwt_ref, sw_ref)
    h_ref[...] = jnp.concatenate(
        [h, jnp.zeros_like(h)], axis=1)
    _accum_stats(stats_ref, h)


def _bn1_body(h_ref, stats_ref, gamma_ref, beta_ref, out_ref, *, n, dh):
    scale, shift = _bn_affine(stats_ref, gamma_ref, beta_ref, n)
    hn = scale * h_ref[...][:, :dh] + shift
    out_ref[...] = jnp.concatenate([hn, jnp.zeros_like(hn)], axis=1)


def _layer2_body(hn_ref, aa_ref, ab_ref, bwt_ref, sw_ref,
                 h2_ref, stats2_ref, *, dh):
    y = (hn_ref[...] + aa_ref[0] + ab_ref[0])[:, :dh]
    h2 = _kan_core(y, bwt_ref, sw_ref)
    h2_ref[...] = h2
    _accum_stats(stats2_ref, h2)


def _layer3_body(h_ref, stats2_ref, gamma_ref, beta_ref, bwt_ref, sw_ref,
                 out_ref, *, n):
    scale, shift = _bn_affine(stats2_ref, gamma_ref, beta_ref, n)
    y = scale * h_ref[...] + shift
    h3 = _kan_core(y, bwt_ref, sw_ref)
    m = jnp.max(h3, axis=1, keepdims=True)
    e = jnp.exp(h3 - m)
    lse = jnp.log(jnp.sum(e, axis=1, keepdims=True))
    out_ref[...] = h3 - m - lse


def _whole(shape):
    return pl.BlockSpec(shape, lambda i: (0,) * len(shape))


_K = 80
_NBUF = 2


def _sc_segsum(xmat, srcr, dstr, zeros, nchunk):
    n = xmat.shape[0] - 8
    d = xmat.shape[1]
    info = plsc.get_sparse_core_info()
    nc, ns = info.num_cores, info.num_subcores
    rpt = (n // ns) // 8 * 8
    tail = n - rpt * ns

    mesh = plsc.VectorSubcoreMesh(core_axis_name="c", subcore_axis_name="s")

    @functools.partial(
        pl.kernel, mesh=mesh,
        out_type=jax.ShapeDtypeStruct((nc, n, d), jnp.float32),
        scratch_types=[
            pltpu.VMEM((nchunk // 2, _K), jnp.int32),
            pltpu.VMEM((nchunk // 2, _K), jnp.int32),
            pltpu.VMEM((_K,), jnp.int32),
            pltpu.VMEM((_K,), jnp.int32),
            pltpu.VMEM((_K,), jnp.int32),
            pltpu.VMEM((_K,), jnp.int32),
            pltpu.VMEM((_K, d), jnp.float32),
            pltpu.VMEM((_K, d), jnp.float32),
            pltpu.VMEM_SHARED((n, d), jnp.float32),
            pltpu.SemaphoreType.DMA,
            pltpu.SemaphoreType.DMA,
            pltpu.SemaphoreType.DMA,
            pltpu.SemaphoreType.DMA,
        ])
    def seg_kernel(x_hbm, srcr_hbm, dstr_hbm, z_hbm, out_hbm,
                   sidx, didx, scur0, scur1, dcur0, dcur1, rows0, rows1,
                   acc_sh, g0, g1, s0, s1):
        c = lax.axis_index("c")
        s = lax.axis_index("s")
        wid = s * nc + c
        r0 = s * rpt
        is_last = s == (ns - 1)

        pltpu.sync_copy(z_hbm.at[pl.ds(r0, rpt)], acc_sh.at[pl.ds(r0, rpt)])
        if tail:
            @pl.when(is_last)
            def _():
                pltpu.sync_copy(z_hbm.at[pl.ds(rpt * ns, tail)],
                                acc_sh.at[pl.ds(rpt * ns, tail)])
        plsc.subcore_barrier()

        npc = nchunk // 2

        for half in range(2):
            cb = half * npc
            pltpu.sync_copy(srcr_hbm.at[wid, pl.ds(cb, npc)], sidx)
            pltpu.sync_copy(dstr_hbm.at[wid, pl.ds(cb, npc)], didx)

            def stage_idx(j, scur, dcur):
                for i in range(_K // 16):
                    sl = pl.ds(i * 16, 16)
                    scur[sl] = sidx[j, sl]
                    dcur[sl] = didx[j, sl]

            stage_idx(0, scur0, dcur0)
            pltpu.async_copy(x_hbm.at[scur0], rows0, g0)

            def pair(t, carry):
                j0 = 2 * t
                pltpu.make_async_copy(x_hbm.at[scur0], rows0, g0).wait()

                @pl.when(t > 0)
                def _():
                    pltpu.make_async_copy(rows1, acc_sh.at[dcur1], s1).wait()

                stage_idx(j0 + 1, scur1, dcur1)
                pltpu.async_copy(x_hbm.at[scur1], rows1, g1)
                pltpu.async_copy(rows0, acc_sh.at[dcur0], s0, add=True)

                pltpu.make_async_copy(x_hbm.at[scur1], rows1, g1).wait()
                pltpu.make_async_copy(rows0, acc_sh.at[dcur0], s0).wait()

                @pl.when(j0 + 2 < npc)
                def _():
                    stage_idx(j0 + 2, scur0, dcur0)
                    pltpu.async_copy(x_hbm.at[scur0], rows0, g0)

                pltpu.async_copy(rows1, acc_sh.at[dcur1], s1, add=True)
                return carry

            lax.fori_loop(0, npc // 2, pair, 0)
            pltpu.make_async_copy(rows1, acc_sh.at[dcur1], s1).wait()

        plsc.subcore_barrier()

        pltpu.sync_copy(acc_sh.at[pl.ds(r0, rpt)],
                        out_hbm.at[c, pl.ds(r0, rpt)])
        if tail:
            @pl.when(is_last)
            def _():
                pltpu.sync_copy(acc_sh.at[pl.ds(rpt * ns, tail)],
                                out_hbm.at[c, pl.ds(rpt * ns, tail)])

    return seg_kernel(xmat, srcr, dstr, zeros)


def kernel(x, edge_index, batch, base_w1, spline_w1, bn1_gamma, bn1_beta,
           base_w2, spline_w2, bn2_gamma, bn2_beta, base_w3, spline_w3):
    del batch
    n, d_in = x.shape
    dh = base_w2.shape[1]
    c_out = base_w3.shape[0]
    src = edge_index[0]
    dst = edge_index[1]
    e = src.shape[0]

    nw = 32
    nchunk = -(-e // (nw * _K))
    r = 2 * _NBUF
    if nchunk % r:
        nchunk += r - nchunk % r
    pad_len = nw * _K * nchunk - e
    src_p = jnp.concatenate([src, jnp.full((pad_len,), n, jnp.int32)])
    dst_p = jnp.concatenate([dst, jnp.zeros((pad_len,), jnp.int32)])
    srcr = src_p.reshape(nchunk, nw, _K).transpose(1, 0, 2)
    dstr = dst_p.reshape(nchunk, nw, _K).transpose(1, 0, 2)

    zeros_in = jnp.zeros((n, d_in), jnp.float32)
    zrows = jnp.zeros((8, d_in), jnp.float32)
    xp = jnp.concatenate([x, zrows])

    bwt1 = base_w1.T
    swt1 = jnp.transpose(spline_w1, (2, 1, 0))
    bwt2 = base_w2.T
    swt2 = jnp.transpose(spline_w2, (2, 1, 0))
    bwt3 = base_w3.T
    swt3 = jnp.transpose(spline_w3, (2, 1, 0))
    g1 = bn1_gamma.reshape(1, dh)
    b1 = bn1_beta.reshape(1, dh)
    g2 = bn2_gamma.reshape(1, dh)
    b2 = bn2_beta.reshape(1, dh)

    agg1 = _sc_segsum(xp, srcr, dstr, zeros_in, nchunk)

    tn = 400
    grid = (n // tn,)

    def stacked(d):
        return pl.BlockSpec((1, tn, d), lambda i: (0, i, 0))

    def stacked1(d):
        return pl.BlockSpec((1, tn, d), lambda i: (1, i, 0))

    h1, st1 = pl.pallas_call(
        _layer1_body,
        grid=grid,
        in_specs=[
            pl.BlockSpec((tn, d_in), lambda i: (i, 0)),
            stacked(d_in), stacked1(d_in),
            _whole(bwt1.shape), _whole(swt1.shape),
        ],
        out_specs=[
            pl.BlockSpec((tn, 2 * dh), lambda i: (i, 0)),
            pl.BlockSpec((2, dh), lambda i: (0, 0)),
        ],
        out_shape=[
            jax.ShapeDtypeStruct((n, 2 * dh), jnp.float32),
            jax.ShapeDtypeStruct((2, dh), jnp.float32),
        ],
    )(x, agg1, agg1, bwt1, swt1)

    h1n = pl.pallas_call(
        functools.partial(_bn1_body, n=n, dh=dh),
        grid=grid,
        in_specs=[
            pl.BlockSpec((tn, 2 * dh), lambda i: (i, 0)),
            _whole((2, dh)), _whole((1, dh)), _whole((1, dh)),
        ],
        out_specs=pl.BlockSpec((tn, 2 * dh), lambda i: (i, 0)),
        out_shape=jax.ShapeDtypeStruct((n, 2 * dh), jnp.float32),
    )(h1, st1, g1, b1)

    agg2 = _sc_segsum(jnp.concatenate([h1n, zrows]), srcr, dstr, zeros_in, nchunk)

    h2, st2 = pl.pallas_call(
        functools.partial(_layer2_body, dh=dh),
        grid=grid,
        in_specs=[
            pl.BlockSpec((tn, 2 * dh), lambda i: (i, 0)),
            stacked(2 * dh), stacked1(2 * dh),
            _whole(bwt2.shape), _whole(swt2.shape),
        ],
        out_specs=[
            pl.BlockSpec((tn, dh), lambda i: (i, 0)),
            pl.BlockSpec((2, dh), lambda i: (0, 0)),
        ],
        out_shape=[
            jax.ShapeDtypeStruct((n, dh), jnp.float32),
            jax.ShapeDtypeStruct((2, dh), jnp.float32),
        ],
    )(h1n, agg2, agg2, bwt2, swt2)

    out = pl.pallas_call(
        functools.partial(_layer3_body, n=n),
        grid=grid,
        in_specs=[
            pl.BlockSpec((tn, dh), lambda i: (i, 0)),
            _whole((2, dh)), _whole((1, dh)), _whole((1, dh)),
            _whole(bwt3.shape), _whole(swt3.shape),
        ],
        out_specs=pl.BlockSpec((tn, c_out), lambda i: (i, 0)),
        out_shape=jax.ShapeDtypeStruct((n, c_out), jnp.float32),
    )(h2, st2, g2, b2, bwt3, swt3)

    return out


# speedup vs baseline: 1.3663x; 1.3663x over previous
import functools

import jax
import jax.numpy as jnp
from jax import lax
from jax.experimental import pallas as pl
from jax.experimental.pallas import tpu as pltpu
from jax.experimental.pallas import tpu_sc as plsc

_GRID_SIZE = 5
_SPLINE_ORDER = 3
_NB = _GRID_SIZE + _SPLINE_ORDER
_H = 2.0 / _GRID_SIZE
_G0 = -1.0 - _SPLINE_ORDER * _H
_NKNOTS = _GRID_SIZE + 2 * _SPLINE_ORDER + 1
_EPS = 1e-5


def _bspline_bases(y):
    g = [_G0 + _H * j for j in range(_NKNOTS)]
    b = [((y >= g[j]) & (y < g[j + 1])).astype(y.dtype) for j in range(_NKNOTS - 1)]
    for k in range(1, _SPLINE_ORDER + 1):
        inv = 1.0 / (_H * k)
        nb = []
        for j in range(len(b) - 1):
            left = (y - g[j]) * inv
            right = (g[j + k + 1] - y) * inv
            nb.append(left * b[j] + right * b[j + 1])
        b = nb
    return b


def _kan_core(y, bwt_ref, sw_ref):
    acc = jnp.dot(y * jax.nn.sigmoid(y), bwt_ref[...],
                  preferred_element_type=jnp.float32)
    bases = _bspline_bases(y)
    for b in range(_NB):
        acc = acc + jnp.dot(bases[b], sw_ref[b],
                            preferred_element_type=jnp.float32)
    return acc


def _accum_stats(stats_ref, h):
    st = jnp.concatenate(
        [jnp.sum(h, axis=0, keepdims=True),
         jnp.sum(h * h, axis=0, keepdims=True)], axis=0)

    @pl.when(pl.program_id(0) == 0)
    def _():
        stats_ref[...] = st

    @pl.when(pl.program_id(0) != 0)
    def _():
        stats_ref[...] = stats_ref[...] + st


def _bn_affine(stats_ref, gamma_ref, beta_ref, n):
    mu = stats_ref[0:1, :] * (1.0 / n)
    ex2 = stats_ref[1:2, :] * (1.0 / n)
    var = ex2 - mu * mu
    scale = gamma_ref[...] * lax.rsqrt(var + _EPS)
    shift = beta_ref[...] - mu * scale
    return scale, shift


def _layer1_body(x_ref, aa_ref, ab_ref, bwt_ref, sw_ref, h_ref, stats_ref):
    y = x_ref[...] + aa_ref[0] + ab_ref[0]
    h = _kan_core(y, bwt_ref, sw_ref)
    h_ref[...] = jnp.concatenate(
        [h, jnp.zeros_like(h)], axis=1)
    _accum_stats(stats_ref, h)


def _bn1_body(h_ref, stats_ref, gamma_ref, beta_ref, out_ref, *, n, dh):
    scale, shift = _bn_affine(stats_ref, gamma_ref, beta_ref, n)
    hn = scale * h_ref[...][:, :dh] + shift
    out_ref[...] = jnp.concatenate([hn, jnp.zeros_like(hn)], axis=1)


def _layer2_body(hn_ref, aa_ref, ab_ref, bwt_ref, sw_ref,
                 h2_ref, stats2_ref, *, dh):
    y = (hn_ref[...] + aa_ref[0] + ab_ref[0])[:, :dh]
    h2 = _kan_core(y, bwt_ref, sw_ref)
    h2_ref[...] = h2
    _accum_stats(stats2_ref, h2)


def _layer3_body(h_ref, stats2_ref, gamma_ref, beta_ref, bwt_ref, sw_ref,
                 out_ref, *, n):
    scale, shift = _bn_affine(stats2_ref, gamma_ref, beta_ref, n)
    y = scale * h_ref[...] + shift
    h3 = _kan_core(y, bwt_ref, sw_ref)
    m = jnp.max(h3, axis=1, keepdims=True)
    e = jnp.exp(h3 - m)
    lse = jnp.log(jnp.sum(e, axis=1, keepdims=True))
    out_ref[...] = h3 - m - lse


def _whole(shape):
    return pl.BlockSpec(shape, lambda i: (0,) * len(shape))


_K = 80


def _sc_segsum(xmat, src, dst, zeros):
    n, d = xmat.shape
    e = src.shape[0]
    info = plsc.get_sparse_core_info()
    nc, ns = info.num_cores, info.num_subcores
    nw = nc * ns
    ew = e // nw
    nchunk = ew // _K
    rpt = (n // ns) // 8 * 8
    tail = n - rpt * ns

    mesh = plsc.VectorSubcoreMesh(core_axis_name="c", subcore_axis_name="s")

    @functools.partial(
        pl.kernel, mesh=mesh,
        out_type=jax.ShapeDtypeStruct((nc, n, d), jnp.float32),
        scratch_types=[
            pltpu.VMEM((_K,), jnp.int32),
            pltpu.VMEM((_K,), jnp.int32),
            pltpu.VMEM((_K, d), jnp.float32),
            pltpu.VMEM_SHARED((n, d), jnp.float32),
            pltpu.SemaphoreType.DMA,
        ])
    def seg_kernel(x_hbm, src_hbm, dst_hbm, z_hbm, out_hbm,
                   sidx, didx, rows, acc_sh, sem):
        c = lax.axis_index("c")
        s = lax.axis_index("s")
        wid = s * nc + c
        r0 = s * rpt
        is_last = s == (ns - 1)

        pltpu.sync_copy(z_hbm.at[pl.ds(r0, rpt)], acc_sh.at[pl.ds(r0, rpt)])
        if tail:
            @pl.when(is_last)
            def _():
                pltpu.sync_copy(z_hbm.at[pl.ds(rpt * ns, tail)],
                                acc_sh.at[pl.ds(rpt * ns, tail)])
        plsc.subcore_barrier()

        base = wid * ew

        def body(i, carry):
            off = pl.multiple_of(base + i * _K, 8)
            pltpu.sync_copy(src_hbm.at[pl.ds(off, _K)], sidx)
            pltpu.async_copy(x_hbm.at[sidx], rows, sem).wait()
            pltpu.sync_copy(dst_hbm.at[pl.ds(off, _K)], didx)
            pltpu.sync_copy(rows, acc_sh.at[didx], add=True)
            return carry

        lax.fori_loop(0, nchunk, body, 0)
        plsc.subcore_barrier()

        pltpu.sync_copy(acc_sh.at[pl.ds(r0, rpt)],
                        out_hbm.at[c, pl.ds(r0, rpt)])
        if tail:
            @pl.when(is_last)
            def _():
                pltpu.sync_copy(acc_sh.at[pl.ds(rpt * ns, tail)],
                                out_hbm.at[c, pl.ds(rpt * ns, tail)])

    return seg_kernel(xmat, src, dst, zeros)


def kernel(x, edge_index, batch, base_w1, spline_w1, bn1_gamma, bn1_beta,
           base_w2, spline_w2, bn2_gamma, bn2_beta, base_w3, spline_w3):
    del batch
    n, d_in = x.shape
    dh = base_w2.shape[1]
    c_out = base_w3.shape[0]
    src = edge_index[0]
    dst = edge_index[1]

    zeros_in = jnp.zeros((n, d_in), jnp.float32)

    bwt1 = base_w1.T
    swt1 = jnp.transpose(spline_w1, (2, 1, 0))
    bwt2 = base_w2.T
    swt2 = jnp.transpose(spline_w2, (2, 1, 0))
    bwt3 = base_w3.T
    swt3 = jnp.transpose(spline_w3, (2, 1, 0))
    g1 = bn1_gamma.reshape(1, dh)
    b1 = bn1_beta.reshape(1, dh)
    g2 = bn2_gamma.reshape(1, dh)
    b2 = bn2_beta.reshape(1, dh)

    agg1 = _sc_segsum(x, src, dst, zeros_in)

    tn = 400
    grid = (n // tn,)

    def stacked(d):
        return pl.BlockSpec((1, tn, d), lambda i: (0, i, 0))

    def stacked1(d):
        return pl.BlockSpec((1, tn, d), lambda i: (1, i, 0))

    h1, st1 = pl.pallas_call(
        _layer1_body,
        grid=grid,
        in_specs=[
            pl.BlockSpec((tn, d_in), lambda i: (i, 0)),
            stacked(d_in), stacked1(d_in),
            _whole(bwt1.shape), _whole(swt1.shape),
        ],
        out_specs=[
            pl.BlockSpec((tn, 2 * dh), lambda i: (i, 0)),
            pl.BlockSpec((2, dh), lambda i: (0, 0)),
        ],
        out_shape=[
            jax.ShapeDtypeStruct((n, 2 * dh), jnp.float32),
            jax.ShapeDtypeStruct((2, dh), jnp.float32),
        ],
    )(x, agg1, agg1, bwt1, swt1)

    h1n = pl.pallas_call(
        functools.partial(_bn1_body, n=n, dh=dh),
        grid=grid,
        in_specs=[
            pl.BlockSpec((tn, 2 * dh), lambda i: (i, 0)),
            _whole((2, dh)), _whole((1, dh)), _whole((1, dh)),
        ],
        out_specs=pl.BlockSpec((tn, 2 * dh), lambda i: (i, 0)),
        out_shape=jax.ShapeDtypeStruct((n, 2 * dh), jnp.float32),
    )(h1, st1, g1, b1)

    agg2 = _sc_segsum(h1n, src, dst, zeros_in)

    h2, st2 = pl.pallas_call(
        functools.partial(_layer2_body, dh=dh),
        grid=grid,
        in_specs=[
            pl.BlockSpec((tn, 2 * dh), lambda i: (i, 0)),
            stacked(2 * dh), stacked1(2 * dh),
            _whole(bwt2.shape), _whole(swt2.shape),
        ],
        out_specs=[
            pl.BlockSpec((tn, dh), lambda i: (i, 0)),
            pl.BlockSpec((2, dh), lambda i: (0, 0)),
        ],
        out_shape=[
            jax.ShapeDtypeStruct((n, dh), jnp.float32),
            jax.ShapeDtypeStruct((2, dh), jnp.float32),
        ],
    )(h1n, agg2, agg2, bwt2, swt2)

    out = pl.pallas_call(
        functools.partial(_layer3_body, n=n),
        grid=grid,
        in_specs=[
            pl.BlockSpec((tn, dh), lambda i: (i, 0)),
            _whole((2, dh)), _whole((1, dh)), _whole((1, dh)),
            _whole(bwt3.shape), _whole(swt3.shape),
        ],
        out_specs=pl.BlockSpec((tn, c_out), lambda i: (i, 0)),
        out_shape=jax.ShapeDtypeStruct((n, c_out), jnp.float32),
    )(h2, st2, g2, b2, bwt3, swt3)

    return out
